# baseline (speedup 1.0000x reference)
import functools

import jax
import jax.numpy as jnp
from jax import lax
from jax.experimental import pallas as pl
from jax.experimental.pallas import tpu as pltpu
from jax.experimental.pallas import tpu_sc as plsc

B = 16384
IN_DIM = 100
N_CAT = 26
VOCAB = 100000
EMB = 3
CAT_W = N_CAT * EMB
OUT_W = CAT_W + IN_DIM - N_CAT

NC, NS = 2, 16
NW = NC * NS
R = B // NW
C = 256
NCH = R // C
MC = C * N_CAT
NIDX = MC // 128

_mesh = plsc.VectorSubcoreMesh(core_axis_name="c", subcore_axis_name="s")


@functools.partial(
    pl.kernel,
    out_type=jax.ShapeDtypeStruct((B, OUT_W), jnp.float32),
    mesh=_mesh,
    compiler_params=pltpu.CompilerParams(
        use_tc_tiling_on_sc=False, needs_layout_passes=False),
    scratch_types=[
        pltpu.VMEM((C, IN_DIM), jnp.float32),
        pltpu.VMEM((NIDX, 128), jnp.int32),
        pltpu.VMEM((EMB, NIDX, 128), jnp.float32),
        pltpu.VMEM((C, OUT_W), jnp.float32),
        pltpu.SemaphoreType.DMA,
        pltpu.SemaphoreType.DMA,
        pltpu.SemaphoreType.DMA,
    ],
)
def _emb_kernel(xb_hbm, t0_hbm, t1_hbm, t2_hbm, out_hbm,
                xv, idxv, gq, ov, lsem, gsem, osem):
    wid = lax.axis_index("s") * NC + lax.axis_index("c")
    lane = lax.iota(jnp.int32, 16)

    rep = []
    for c0 in (0, 16, 32, 48, 62):
        w = lane + c0
        m3 = lax.shift_right_logical(w * 21846, 16)
        rep.append((c0, m3, w - m3 * 3))

    for ch in range(NCH):
        base = wid * R + ch * C
        pltpu.async_copy(xb_hbm.at[pl.ds(base, C), :], xv, lsem).wait()

        @pl.loop(0, NIDX)
        def _build(v):
            j = lax.shift_right_logical(v, 1)
            r0 = lax.bitwise_and(v, 1) * 128
            for u in range(8):
                r = lane + (r0 + u * 16)
                jv = jnp.full((16,), 0, dtype=jnp.int32) + j
                bits = plsc.load_gather(xv, [r, jv])
                idxv[v, pl.ds(u * 16, 16)] = plsc.bitcast(bits, jnp.int32)

        copies = []
        for v in range(NIDX):
            j = v // 2
            iv = idxv.at[v]
            for k, tk in enumerate((t0_hbm, t1_hbm, t2_hbm)):
                copies.append(pltpu.async_copy(
                    tk.at[j].at[iv], gq.at[k, v], gsem))

        @pl.loop(0, C)
        def _convert(r):
            for c in (26, 42, 58, 74, 84):
                bits = xv[r, pl.ds(c, 16)]
                ov[r, pl.ds(c + 52, 16)] = (
                    plsc.bitcast(bits, jnp.int32).astype(jnp.float32))

        for cp in copies:
            cp.wait()

        @pl.loop(0, C)
        def _repack(r):
            rh = lax.shift_right_logical(r, 7)
            rb = lax.bitwise_and(r, 127)
            for c0, m3, k in rep:
                a = m3 * 2 + rh
                b = jnp.full((16,), 0, dtype=jnp.int32) + rb
                ov[r, pl.ds(c0, 16)] = plsc.load_gather(gq, [k, a, b])

        pltpu.async_copy(ov, out_hbm.at[pl.ds(base, C), :], osem).wait()


def kernel(x, tables):
    xb = lax.bitcast_convert_type(x, jnp.float32)
    planes = [tables[:, :, k] for k in range(EMB)]
    return _emb_kernel(xb, *planes)


# speedup vs baseline: 1.1644x; 1.1644x over previous
import functools

import jax
import jax.numpy as jnp
from jax import lax
from jax.experimental import pallas as pl
from jax.experimental.pallas import tpu as pltpu
from jax.experimental.pallas import tpu_sc as plsc

B = 16384
IN_DIM = 100
N_CAT = 26
VOCAB = 100000
EMB = 3
CAT_W = N_CAT * EMB
OUT_W = CAT_W + IN_DIM - N_CAT

NC, NS = 2, 16
NW = NC * NS
R = B // NW
C = 128
NCH = R // C
MC = C * N_CAT
NIDX = MC // 128

_mesh = plsc.VectorSubcoreMesh(core_axis_name="c", subcore_axis_name="s")


@functools.partial(
    pl.kernel,
    out_type=jax.ShapeDtypeStruct((B, OUT_W), jnp.float32),
    mesh=_mesh,
    compiler_params=pltpu.CompilerParams(
        use_tc_tiling_on_sc=False, needs_layout_passes=False),
    scratch_types=[
        pltpu.VMEM((R, IN_DIM), jnp.int32),
        pltpu.VMEM((2, NIDX, 128), jnp.int32),
        pltpu.VMEM((2, EMB, NIDX, 128), jnp.float32),
        pltpu.VMEM((2, C, OUT_W), jnp.float32),
        pltpu.SemaphoreType.DMA,
        pltpu.SemaphoreType.DMA,
        pltpu.SemaphoreType.DMA,
    ],
)
def _emb_kernel(x_hbm, t0_hbm, t1_hbm, t2_hbm, out_hbm,
                xv, idxv, gq, ov, lsem, gsem, osem):
    wid = lax.axis_index("s") * NC + lax.axis_index("c")
    base = wid * R
    lane = lax.iota(jnp.int32, 16)

    rep = []
    for c0 in (0, 16, 32, 48, 62):
        w = lane + c0
        m3 = lax.shift_right_logical(w * 21846, 16)
        rep.append((c0, m3, w - m3 * 3))

    pltpu.async_copy(x_hbm.at[pl.ds(base, R), :], xv, lsem).wait()

    def build_and_fire(ch, sl):
        r0 = ch * C

        @pl.loop(0, NIDX)
        def _build(v):
            for u in range(8):
                m = lane + (v * 128 + u * 16)
                r = lax.shift_right_logical(m * 20165, 19)
                j = m - r * N_CAT
                xi = plsc.load_gather(xv, [r + r0, j])
                idxv[sl, v, pl.ds(u * 16, 16)] = xi + j * VOCAB

        copies = []
        for v in range(NIDX):
            iv = idxv.at[sl, v]
            copies.append(pltpu.async_copy(t0_hbm.at[iv], gq.at[sl, 0, v], gsem))
            copies.append(pltpu.async_copy(t1_hbm.at[iv], gq.at[sl, 1, v], gsem))
            copies.append(pltpu.async_copy(t2_hbm.at[iv], gq.at[sl, 2, v], gsem))
        return copies

    def convert(ch, sl):
        r0 = ch * C

        @pl.loop(0, C)
        def _convert(r):
            for c in (26, 42, 58, 74, 84):
                ov[sl, r, pl.ds(c + 52, 16)] = (
                    xv[r + r0, pl.ds(c, 16)].astype(jnp.float32))

    def repack_and_out(ch, sl, copies):
        for cp in copies:
            cp.wait()

        @pl.loop(0, C)
        def _repack(r):
            r26 = r * N_CAT
            for c0, m3, k in rep:
                m = m3 + r26
                a = lax.shift_right_logical(m, 7)
                b = lax.bitwise_and(m, 127)
                ov[sl, r, pl.ds(c0, 16)] = plsc.load_gather(gq, [
                    jnp.full((16,), sl, dtype=jnp.int32), k, a, b])

        return pltpu.async_copy(
            ov.at[sl], out_hbm.at[pl.ds(base + ch * C, C), :], osem)

    copies = build_and_fire(0, 0)
    convert(0, 0)
    out_cp = None
    for ch in range(NCH):
        sl = ch & 1
        if ch + 1 < NCH:
            nxt = build_and_fire(ch + 1, 1 - sl)
            convert(ch + 1, 1 - sl)
        if out_cp is not None:
            out_cp.wait()
        out_cp = repack_and_out(ch, sl, copies)
        if ch + 1 < NCH:
            copies = nxt
    out_cp.wait()


def kernel(x, tables):
    planes = [tables[:, :, k].reshape(N_CAT * VOCAB) for k in range(EMB)]
    return _emb_kernel(x, *planes)
